# baseline (speedup 1.0000x reference)
import jax
import jax.numpy as jnp
from jax.experimental import pallas as pl
from jax.experimental.pallas import tpu as pltpu

N_TOK = 16384
D_MODEL = 4096
N_EXPERTS = 64
MAX_K = 8
HIGH_CONF = 0.6
MID_CONF = 0.4

BT = 1024


def _router_kernel(x_ref, w_ref, idx_ref, wgt_ref, k_ref, aux_ref,
                   cnt_ref, psum_ref):
    i = pl.program_id(0)
    nsteps = pl.num_programs(0)

    x = x_ref[...]
    w = w_ref[...]
    logits = jax.lax.dot_general(
        w, x, (((1,), (1,)), ((), ())),
        preferred_element_type=jnp.float32)

    m = jnp.max(logits, axis=0, keepdims=True)
    e = jnp.exp(logits - m)
    s = jnp.sum(e, axis=0, keepdims=True)
    probs = e / s

    erow = jax.lax.broadcasted_iota(jnp.int32, (N_EXPERTS, BT), 0)
    srow = jax.lax.broadcasted_iota(jnp.int32, (MAX_K, BT), 0)

    work = probs
    vals = jnp.zeros((MAX_K, BT), jnp.float32)
    idxs = jnp.zeros((MAX_K, BT), jnp.int32)
    for j in range(MAX_K):
        mj = jnp.max(work, axis=0, keepdims=True)
        ij = jnp.min(jnp.where(work == mj, erow, N_EXPERTS),
                     axis=0, keepdims=True)
        vals = jnp.where(srow == j, mj, vals)
        idxs = jnp.where(srow == j, ij, idxs)
        work = jnp.where(erow == ij, -jnp.inf, work)

    top1 = vals[0:1, :]
    k = jnp.where(top1 >= HIGH_CONF, 1,
                  jnp.where(top1 >= MID_CONF, 2, MAX_K)).astype(jnp.int32)
    valid = srow < k
    wgt = vals * valid.astype(jnp.float32)
    wsum = jnp.clip(jnp.sum(wgt, axis=0, keepdims=True), 1e-9, None)

    idx_ref[...] = jnp.where(valid, idxs, -1).T
    wgt_ref[...] = (wgt / wsum).T
    k_ref[...] = k.T

    top1_idx = idxs[0:1, :]
    cnt_blk = jnp.sum((erow == top1_idx).astype(jnp.float32),
                      axis=1, keepdims=True)
    p_blk = jnp.sum(probs, axis=1, keepdims=True)

    @pl.when(i == 0)
    def _init():
        cnt_ref[...] = cnt_blk
        psum_ref[...] = p_blk

    @pl.when(i > 0)
    def _acc():
        cnt_ref[...] += cnt_blk
        psum_ref[...] += p_blk

    @pl.when(i == nsteps - 1)
    def _fin():
        prod = cnt_ref[...] * psum_ref[...] * (N_EXPERTS / (N_TOK * N_TOK))
        aux_ref[...] = jnp.sum(prod, axis=0, keepdims=True)


def kernel(x, W):
    grid = (N_TOK // BT,)
    indices, weights, k1, aux = pl.pallas_call(
        _router_kernel,
        grid=grid,
        in_specs=[
            pl.BlockSpec((BT, D_MODEL), lambda i: (i, 0)),
            pl.BlockSpec((N_EXPERTS, D_MODEL), lambda i: (0, 0)),
        ],
        out_specs=[
            pl.BlockSpec((BT, MAX_K), lambda i: (i, 0)),
            pl.BlockSpec((BT, MAX_K), lambda i: (i, 0)),
            pl.BlockSpec((BT, 1), lambda i: (i, 0)),
            pl.BlockSpec((1, 1), lambda i: (0, 0)),
        ],
        out_shape=[
            jax.ShapeDtypeStruct((N_TOK, MAX_K), jnp.int32),
            jax.ShapeDtypeStruct((N_TOK, MAX_K), jnp.float32),
            jax.ShapeDtypeStruct((N_TOK, 1), jnp.int32),
            jax.ShapeDtypeStruct((1, 1), jnp.float32),
        ],
        scratch_shapes=[
            pltpu.VMEM((N_EXPERTS, 1), jnp.float32),
            pltpu.VMEM((N_EXPERTS, 1), jnp.float32),
        ],
        compiler_params=pltpu.CompilerParams(
            dimension_semantics=("arbitrary",)),
    )(x, W)
    return indices, weights, k1.reshape(N_TOK), aux.reshape(())


# speedup vs baseline: 1.2969x; 1.2969x over previous
import jax
import jax.numpy as jnp
from jax.experimental import pallas as pl
from jax.experimental.pallas import tpu as pltpu

N_TOK = 16384
D_MODEL = 4096
N_EXPERTS = 64
MAX_K = 8
HIGH_CONF = 0.6
MID_CONF = 0.4

BT = 1024


def _router_kernel(x_ref, w_ref, idx_ref, wgt_ref, k_ref, aux_ref,
                   cnt_ref, psum_ref):
    i = pl.program_id(0)
    nsteps = pl.num_programs(0)

    x = x_ref[...]
    w = w_ref[...]
    logits = jax.lax.dot_general(
        w, x, (((1,), (1,)), ((), ())),
        preferred_element_type=jnp.float32)

    m = jnp.max(logits, axis=0, keepdims=True)
    e = jnp.exp(logits - m)
    s = jnp.sum(e, axis=0, keepdims=True)
    probs = e / s

    erow = jax.lax.broadcasted_iota(jnp.int32, (N_EXPERTS, BT), 0)
    srow = jax.lax.broadcasted_iota(jnp.int32, (MAX_K, BT), 0)

    work = probs
    vals = jnp.zeros((MAX_K, BT), jnp.float32)
    idxs = jnp.zeros((MAX_K, BT), jnp.int32)
    for j in range(MAX_K):
        mj = jnp.max(work, axis=0, keepdims=True)
        ij = jnp.min(jnp.where(work == mj, erow, N_EXPERTS),
                     axis=0, keepdims=True)
        vals = jnp.where(srow == j, mj, vals)
        idxs = jnp.where(srow == j, ij, idxs)
        work = jnp.where(erow == ij, -jnp.inf, work)

    top1 = vals[0:1, :]
    k = jnp.where(top1 >= HIGH_CONF, 1,
                  jnp.where(top1 >= MID_CONF, 2, MAX_K)).astype(jnp.int32)
    valid = srow < k
    wgt = vals * valid.astype(jnp.float32)
    wsum = jnp.clip(jnp.sum(wgt, axis=0, keepdims=True), 1e-9, None)

    idx_ref[...] = jnp.where(valid, idxs, -1)
    wgt_ref[...] = wgt / wsum
    k_ref[...] = k

    top1_idx = idxs[0:1, :]
    cnt_blk = jnp.sum((erow == top1_idx).astype(jnp.float32),
                      axis=1, keepdims=True)
    p_blk = jnp.sum(probs, axis=1, keepdims=True)

    @pl.when(i == 0)
    def _init():
        cnt_ref[...] = cnt_blk
        psum_ref[...] = p_blk

    @pl.when(i > 0)
    def _acc():
        cnt_ref[...] += cnt_blk
        psum_ref[...] += p_blk

    @pl.when(i == nsteps - 1)
    def _fin():
        prod = cnt_ref[...] * psum_ref[...] * (N_EXPERTS / (N_TOK * N_TOK))
        aux_ref[...] = jnp.sum(prod, axis=0, keepdims=True)


def kernel(x, W):
    grid = (N_TOK // BT,)
    idxT, wgtT, k1, aux = pl.pallas_call(
        _router_kernel,
        grid=grid,
        in_specs=[
            pl.BlockSpec((BT, D_MODEL), lambda i: (i, 0)),
            pl.BlockSpec((N_EXPERTS, D_MODEL), lambda i: (0, 0)),
        ],
        out_specs=[
            pl.BlockSpec((MAX_K, BT), lambda i: (0, i)),
            pl.BlockSpec((MAX_K, BT), lambda i: (0, i)),
            pl.BlockSpec((1, BT), lambda i: (0, i)),
            pl.BlockSpec((1, 1), lambda i: (0, 0)),
        ],
        out_shape=[
            jax.ShapeDtypeStruct((MAX_K, N_TOK), jnp.int32),
            jax.ShapeDtypeStruct((MAX_K, N_TOK), jnp.float32),
            jax.ShapeDtypeStruct((1, N_TOK), jnp.int32),
            jax.ShapeDtypeStruct((1, 1), jnp.float32),
        ],
        scratch_shapes=[
            pltpu.VMEM((N_EXPERTS, 1), jnp.float32),
            pltpu.VMEM((N_EXPERTS, 1), jnp.float32),
        ],
        compiler_params=pltpu.CompilerParams(
            dimension_semantics=("arbitrary",)),
    )(x, W)
    return idxT.T, wgtT.T, k1.reshape(N_TOK), aux.reshape(())
